# baseline (device time: 27684 ns/iter reference)
import jax
import jax.numpy as jnp
from jax import lax
from jax.experimental import pallas as pl
from jax.experimental.pallas import tpu as pltpu

N_DEV = 8
N_TOK = 1024
D_IN = 256
D_OUT = 512
E_TOTAL = 32
E_LOCAL = E_TOTAL // N_DEV
CAP = 25
TOK_PER = N_TOK // N_DEV


def kernel(x, router_W, route_idx, expert_W):
    my = lax.axis_index("i")

    e = route_idx[:, 0]
    oh = jax.nn.one_hot(e, E_TOTAL, dtype=jnp.float32)
    pos = jnp.cumsum(oh, axis=0) - oh
    keep = (jnp.sum(pos * oh, axis=1) < CAP).astype(jnp.float32)
    s = oh * keep[:, None]
    s_local = lax.dynamic_slice(s, (0, my * E_LOCAL), (N_TOK, E_LOCAL))

    def body(x_ref, s_ref, w_ref, out_ref, partial_ref, comm_ref,
             send_sems, recv_sems):
        my_pos = lax.axis_index("i")

        comm_ref[...] = jnp.zeros_like(comm_ref)

        barrier = pltpu.get_barrier_semaphore()
        for d in range(N_DEV):
            @pl.when(my_pos != d)
            def _():
                pl.semaphore_signal(
                    barrier, inc=1,
                    device_id=(d,), device_id_type=pl.DeviceIdType.MESH,
                )
        pl.semaphore_wait(barrier, N_DEV - 1)

        xb = x_ref[...].astype(jnp.bfloat16)
        acc = jnp.zeros((N_TOK, D_OUT), jnp.float32)
        for ee in range(E_LOCAL):
            w = w_ref[ee].astype(jnp.bfloat16)
            acc += s_ref[:, ee:ee + 1] * jnp.dot(
                xb, w, preferred_element_type=jnp.float32)
        partial_ref[...] = acc.astype(jnp.bfloat16)

        for k in range(N_DEV):
            @pl.when(my_pos != k)
            def _():
                rdma = pltpu.make_async_remote_copy(
                    src_ref=partial_ref.at[pl.ds(k * TOK_PER, TOK_PER), :],
                    dst_ref=comm_ref.at[my_pos],
                    send_sem=send_sems.at[k],
                    recv_sem=recv_sems.at[my_pos],
                    device_id=(k,),
                    device_id_type=pl.DeviceIdType.MESH,
                )
                rdma.start()

        for d in range(N_DEV):
            @pl.when(my_pos != d)
            def _():
                recv = pltpu.make_async_remote_copy(
                    src_ref=partial_ref.at[pl.ds(0, TOK_PER), :],
                    dst_ref=comm_ref.at[d],
                    send_sem=send_sems.at[0],
                    recv_sem=recv_sems.at[d],
                    device_id=(d,),
                    device_id_type=pl.DeviceIdType.MESH,
                )
                recv.wait_recv()

        total = partial_ref[pl.ds(my_pos * TOK_PER, TOK_PER), :].astype(
            jnp.float32)
        for d in range(N_DEV):
            total += comm_ref[d].astype(jnp.float32)
        out_ref[...] = total

        for k in range(N_DEV):
            @pl.when(my_pos != k)
            def _():
                send = pltpu.make_async_remote_copy(
                    src_ref=partial_ref.at[pl.ds(k * TOK_PER, TOK_PER), :],
                    dst_ref=comm_ref.at[my_pos],
                    send_sem=send_sems.at[k],
                    recv_sem=recv_sems.at[my_pos],
                    device_id=(k,),
                    device_id_type=pl.DeviceIdType.MESH,
                )
                send.wait_send()

    return pl.pallas_call(
        body,
        out_shape=jax.ShapeDtypeStruct((TOK_PER, D_OUT), jnp.float32),
        in_specs=[
            pl.BlockSpec(memory_space=pltpu.VMEM),
            pl.BlockSpec(memory_space=pltpu.VMEM),
            pl.BlockSpec(memory_space=pltpu.VMEM),
        ],
        out_specs=pl.BlockSpec(memory_space=pltpu.VMEM),
        scratch_shapes=[
            pltpu.VMEM((N_TOK, D_OUT), jnp.bfloat16),
            pltpu.VMEM((N_DEV, TOK_PER, D_OUT), jnp.bfloat16),
            pltpu.SemaphoreType.DMA((N_DEV,)),
            pltpu.SemaphoreType.DMA((N_DEV,)),
        ],
        compiler_params=pltpu.CompilerParams(collective_id=0),
    )(x, s_local, expert_W)


# device time: 19048 ns/iter; 1.4534x vs baseline; 1.4534x over previous
import jax
import jax.numpy as jnp
from jax import lax
from jax.experimental import pallas as pl
from jax.experimental.pallas import tpu as pltpu

N_DEV = 8
N_TOK = 1024
D_IN = 256
D_OUT = 512
E_TOTAL = 32
E_LOCAL = E_TOTAL // N_DEV
CAP = 25
TOK_PER = N_TOK // N_DEV


def kernel(x, router_W, route_idx, expert_W):
    del router_W

    def body(x_ref, e_ref, w_ref, out_ref, partial_ref, comm_ref,
             send_sems, recv_sems):
        my_pos = lax.axis_index("i")

        comm_ref[...] = jnp.zeros_like(comm_ref)

        barrier = pltpu.get_barrier_semaphore()
        for d in range(N_DEV):
            @pl.when(my_pos != d)
            def _():
                pl.semaphore_signal(
                    barrier, inc=1,
                    device_id=(d,), device_id_type=pl.DeviceIdType.MESH,
                )
        pl.semaphore_wait(barrier, N_DEV - 1)

        e = e_ref[...]
        oh = (e == lax.broadcasted_iota(jnp.int32, (N_TOK, E_TOTAL), 1))
        ohb = oh.astype(jnp.bfloat16)
        tri = (lax.broadcasted_iota(jnp.int32, (N_TOK, N_TOK), 0)
               > lax.broadcasted_iota(jnp.int32, (N_TOK, N_TOK), 1))
        pos = jnp.dot(tri.astype(jnp.bfloat16), ohb,
                      preferred_element_type=jnp.float32)
        rank = jnp.sum(pos * oh.astype(jnp.float32), axis=1, keepdims=True)
        keep = rank < CAP

        xb = x_ref[...].astype(jnp.bfloat16)
        acc = jnp.zeros((N_TOK, D_OUT), jnp.float32)
        for ee in range(E_LOCAL):
            m = jnp.logical_and(keep, e == my_pos * E_LOCAL + ee)
            acc += m.astype(jnp.float32) * jnp.dot(
                xb, w_ref[ee].astype(jnp.bfloat16),
                preferred_element_type=jnp.float32)
        partial_ref[...] = acc.astype(jnp.bfloat16)

        for k in range(N_DEV):
            @pl.when(my_pos != k)
            def _():
                rdma = pltpu.make_async_remote_copy(
                    src_ref=partial_ref.at[pl.ds(k * TOK_PER, TOK_PER), :],
                    dst_ref=comm_ref.at[my_pos],
                    send_sem=send_sems.at[k],
                    recv_sem=recv_sems.at[my_pos],
                    device_id=(k,),
                    device_id_type=pl.DeviceIdType.MESH,
                )
                rdma.start()

        for d in range(N_DEV):
            @pl.when(my_pos != d)
            def _():
                recv = pltpu.make_async_remote_copy(
                    src_ref=partial_ref.at[pl.ds(0, TOK_PER), :],
                    dst_ref=comm_ref.at[d],
                    send_sem=send_sems.at[0],
                    recv_sem=recv_sems.at[d],
                    device_id=(d,),
                    device_id_type=pl.DeviceIdType.MESH,
                )
                recv.wait_recv()

        total = partial_ref[pl.ds(my_pos * TOK_PER, TOK_PER), :].astype(
            jnp.float32)
        for d in range(N_DEV):
            total += comm_ref[d].astype(jnp.float32)
        out_ref[...] = total

        for k in range(N_DEV):
            @pl.when(my_pos != k)
            def _():
                send = pltpu.make_async_remote_copy(
                    src_ref=partial_ref.at[pl.ds(k * TOK_PER, TOK_PER), :],
                    dst_ref=comm_ref.at[my_pos],
                    send_sem=send_sems.at[k],
                    recv_sem=recv_sems.at[my_pos],
                    device_id=(k,),
                    device_id_type=pl.DeviceIdType.MESH,
                )
                send.wait_send()

    return pl.pallas_call(
        body,
        out_shape=jax.ShapeDtypeStruct((TOK_PER, D_OUT), jnp.float32),
        in_specs=[
            pl.BlockSpec(memory_space=pltpu.VMEM),
            pl.BlockSpec(memory_space=pltpu.VMEM),
            pl.BlockSpec(memory_space=pltpu.VMEM),
        ],
        out_specs=pl.BlockSpec(memory_space=pltpu.VMEM),
        scratch_shapes=[
            pltpu.VMEM((N_TOK, D_OUT), jnp.bfloat16),
            pltpu.VMEM((N_DEV, TOK_PER, D_OUT), jnp.bfloat16),
            pltpu.SemaphoreType.DMA((N_DEV,)),
            pltpu.SemaphoreType.DMA((N_DEV,)),
        ],
        compiler_params=pltpu.CompilerParams(collective_id=0),
    )(x, route_idx, expert_W)


# device time: 18944 ns/iter; 1.4614x vs baseline; 1.0055x over previous
import jax
import jax.numpy as jnp
from jax import lax
from jax.experimental import pallas as pl
from jax.experimental.pallas import tpu as pltpu

N_DEV = 8
N_TOK = 1024
D_IN = 256
D_OUT = 512
E_TOTAL = 32
E_LOCAL = E_TOTAL // N_DEV
CAP = 25
TOK_PER = N_TOK // N_DEV


def kernel(x, router_W, route_idx, expert_W):
    del router_W

    def body(x_ref, e_ref, w_ref, out_ref, partial_ref, comm_ref, lid_ref,
             send_sems, recv_sems):
        my_pos = lax.axis_index("i")

        comm_ref[...] = jnp.zeros_like(comm_ref)

        barrier = pltpu.get_barrier_semaphore()
        for d in range(N_DEV):
            @pl.when(my_pos != d)
            def _():
                pl.semaphore_signal(
                    barrier, inc=1,
                    device_id=(d,), device_id_type=pl.DeviceIdType.MESH,
                )
        pl.semaphore_wait(barrier, N_DEV - 1)

        e = e_ref[...]
        oh = (e == lax.broadcasted_iota(jnp.int32, (N_TOK, E_TOTAL), 1))
        ohb = oh.astype(jnp.bfloat16)
        tri = (lax.broadcasted_iota(jnp.int32, (N_TOK, N_TOK), 0)
               > lax.broadcasted_iota(jnp.int32, (N_TOK, N_TOK), 1))
        pos = jnp.dot(tri.astype(jnp.bfloat16), ohb,
                      preferred_element_type=jnp.float32)
        rank = jnp.sum(pos * oh.astype(jnp.float32), axis=1, keepdims=True)
        lid_ref[...] = jnp.where(
            jnp.logical_and(rank < CAP,
                            (e // E_LOCAL) == my_pos),
            e - my_pos * E_LOCAL, -1)

        wb = [w_ref[ee].astype(jnp.bfloat16) for ee in range(E_LOCAL)]

        def compute_block(k):
            xk = x_ref[pl.ds(k * TOK_PER, TOK_PER), :].astype(jnp.bfloat16)
            lk = lid_ref[pl.ds(k * TOK_PER, TOK_PER), :]
            accb = jnp.zeros((TOK_PER, D_OUT), jnp.float32)
            for ee in range(E_LOCAL):
                m = (lk == ee).astype(jnp.float32)
                accb += m * jnp.dot(xk, wb[ee],
                                    preferred_element_type=jnp.float32)
            return accb

        for j in range(1, N_DEV):
            k = (my_pos + j) % N_DEV
            accb = compute_block(k)
            partial_ref[pl.ds(k * TOK_PER, TOK_PER), :] = accb.astype(
                jnp.bfloat16)
            rdma = pltpu.make_async_remote_copy(
                src_ref=partial_ref.at[pl.ds(k * TOK_PER, TOK_PER), :],
                dst_ref=comm_ref.at[my_pos],
                send_sem=send_sems.at[k],
                recv_sem=recv_sems.at[my_pos],
                device_id=(k,),
                device_id_type=pl.DeviceIdType.MESH,
            )
            rdma.start()

        total = compute_block(my_pos)

        for j in range(1, N_DEV):
            d = (my_pos + j) % N_DEV
            recv = pltpu.make_async_remote_copy(
                src_ref=partial_ref.at[pl.ds(0, TOK_PER), :],
                dst_ref=comm_ref.at[d],
                send_sem=send_sems.at[my_pos],
                recv_sem=recv_sems.at[d],
                device_id=(d,),
                device_id_type=pl.DeviceIdType.MESH,
            )
            recv.wait_recv()
            total += comm_ref[pl.ds(d, 1), :, :].reshape(
                TOK_PER, D_OUT).astype(jnp.float32)
        out_ref[...] = total

        for j in range(1, N_DEV):
            k = (my_pos + j) % N_DEV
            send = pltpu.make_async_remote_copy(
                src_ref=partial_ref.at[pl.ds(k * TOK_PER, TOK_PER), :],
                dst_ref=comm_ref.at[my_pos],
                send_sem=send_sems.at[k],
                recv_sem=recv_sems.at[my_pos],
                device_id=(k,),
                device_id_type=pl.DeviceIdType.MESH,
            )
            send.wait_send()

    return pl.pallas_call(
        body,
        out_shape=jax.ShapeDtypeStruct((TOK_PER, D_OUT), jnp.float32),
        in_specs=[
            pl.BlockSpec(memory_space=pltpu.VMEM),
            pl.BlockSpec(memory_space=pltpu.VMEM),
            pl.BlockSpec(memory_space=pltpu.VMEM),
        ],
        out_specs=pl.BlockSpec(memory_space=pltpu.VMEM),
        scratch_shapes=[
            pltpu.VMEM((N_TOK, D_OUT), jnp.bfloat16),
            pltpu.VMEM((N_DEV, TOK_PER, D_OUT), jnp.bfloat16),
            pltpu.VMEM((N_TOK, 1), jnp.int32),
            pltpu.SemaphoreType.DMA((N_DEV,)),
            pltpu.SemaphoreType.DMA((N_DEV,)),
        ],
        compiler_params=pltpu.CompilerParams(collective_id=0),
    )(x, route_idx, expert_W)
